# baseline (speedup 1.0000x reference)
import functools
import jax
import jax.numpy as jnp
from jax import lax
from jax.experimental import pallas as pl
from jax.experimental.pallas import tpu as pltpu
from jax.experimental.pallas import tpu_sc as plsc

_H = 32
_NVALS = 12
_N = 256
_R = 8
_NSTRIP = _N // _R
_CHUNK = _R * _N
_HG = 4
_NG = _H // _HG


def _sc_body(dist_hbm, table_hbm, out_hbm, table_v, idx0_v, idx1_v,
             out_v, sem_in0, sem_in1, sem_out):
    nc = 2
    wid = lax.axis_index("s") * nc + lax.axis_index("c")
    pltpu.sync_copy(table_hbm, table_v)

    idx_bufs = (idx0_v, idx1_v)
    sem_ins = (sem_in0, sem_in1)

    pltpu.async_copy(dist_hbm.at[wid, pl.ds(0, _R), :], idx0_v, sem_in0)
    pltpu.async_copy(dist_hbm.at[wid, pl.ds(_R, _R), :], idx1_v, sem_in1)

    def pair_body(cc, _):
        for p in range(2):
            c = 2 * cc + p
            idx_v = idx_bufs[p]
            sem_in = sem_ins[p]

            pltpu.make_async_copy(
                dist_hbm.at[0, pl.ds(0, _R), :], idx_v, sem_in).wait()

            for g in range(_NG):
                @pl.when(c >= 1)
                def _drain():
                    pltpu.make_async_copy(
                        out_v.at[pl.ds(g * _HG, _HG)],
                        out_hbm.at[pl.ds(0, _HG), pl.ds(0, _R), :],
                        sem_out.at[g]).wait()

                ths = [table_v[pl.ds((g * _HG + hh) * 16, 16)]
                       for hh in range(_HG)]

                def gather_body(i, _):
                    r = i >> 4
                    cl = (i & 15) * 16
                    d = idx_v[r, pl.ds(cl, 16)]
                    k = jnp.minimum(jnp.maximum(d + 1, 0), _NVALS - 1)
                    for hh in range(_HG):
                        out_v[g * _HG + hh, r, pl.ds(cl, 16)] = (
                            ths[hh].at[k].get(mode="promise_in_bounds"))
                    return 0

                lax.fori_loop(0, _CHUNK // 16, gather_body, 0, unroll=2)

                pltpu.async_copy(
                    out_v.at[pl.ds(g * _HG, _HG)],
                    out_hbm.at[pl.ds(wid * _H + g * _HG, _HG),
                               pl.ds(c * _R, _R), :],
                    sem_out.at[g])

            nxt = jnp.minimum(c + 2, _NSTRIP - 1)
            pltpu.async_copy(
                dist_hbm.at[wid, pl.ds(nxt * _R, _R), :], idx_v, sem_in)
        return 0

    lax.fori_loop(0, _NSTRIP // 2, pair_body, 0)

    for g in range(_NG):
        pltpu.make_async_copy(
            out_v.at[pl.ds(g * _HG, _HG)],
            out_hbm.at[pl.ds(0, _HG), pl.ds(0, _R), :],
            sem_out.at[g]).wait()
    for p in range(2):
        pltpu.make_async_copy(
            dist_hbm.at[0, pl.ds(0, _R), :], idx_bufs[p], sem_ins[p]).wait()


def kernel(dist_matrix, bias_embedding):
    B, N, _ = dist_matrix.shape
    table_flat = jnp.pad(bias_embedding.T, ((0, 0), (0, 4))).reshape(_H * 16)

    mesh = plsc.VectorSubcoreMesh(core_axis_name="c", subcore_axis_name="s")
    k = functools.partial(
        pl.kernel,
        mesh=mesh,
        out_type=jax.ShapeDtypeStruct((B * _H, N, N), jnp.float32),
        scratch_types=[
            pltpu.VMEM((_H * 16,), jnp.float32),
            pltpu.VMEM((_R, _N), jnp.int32),
            pltpu.VMEM((_R, _N), jnp.int32),
            pltpu.VMEM((_H, _R, _N), jnp.float32),
            pltpu.SemaphoreType.DMA,
            pltpu.SemaphoreType.DMA,
            pltpu.SemaphoreType.DMA((_NG,)),
        ],
        compiler_params=pltpu.CompilerParams(
            needs_layout_passes=False, use_tc_tiling_on_sc=True),
    )(_sc_body)
    return k(dist_matrix, table_flat)


# speedup vs baseline: 2.0272x; 2.0272x over previous
import functools
import jax
import jax.numpy as jnp
from jax import lax
from jax.experimental import pallas as pl
from jax.experimental.pallas import tpu as pltpu
from jax.experimental.pallas import tpu_sc as plsc

_H = 32
_NVALS = 12
_N = 256
_R = 8
_NSTRIP = _N // _R
_CHUNK = _R * _N
_HG = 16
_NG = _H // _HG


def _sc_body(dist_hbm, table_hbm, out_hbm, table_v, idx0_v, idx1_v,
             out_v, sem_in0, sem_in1, sem_out):
    nc = 2
    wid = lax.axis_index("s") * nc + lax.axis_index("c")
    pltpu.sync_copy(table_hbm, table_v)

    idx_bufs = (idx0_v, idx1_v)
    sem_ins = (sem_in0, sem_in1)

    pltpu.async_copy(dist_hbm.at[wid, pl.ds(0, _R), :], idx0_v, sem_in0)
    pltpu.async_copy(dist_hbm.at[wid, pl.ds(_R, _R), :], idx1_v, sem_in1)

    def pair_body(cc, _):
        for p in range(2):
            c = 2 * cc + p
            idx_v = idx_bufs[p]
            sem_in = sem_ins[p]

            pltpu.make_async_copy(
                dist_hbm.at[0, pl.ds(0, _R), :], idx_v, sem_in).wait()

            for g in range(_NG):
                @pl.when(c >= 1)
                def _drain():
                    pltpu.make_async_copy(
                        out_v.at[pl.ds(g * _HG, _HG)],
                        out_hbm.at[pl.ds(0, _HG), pl.ds(0, _R), :],
                        sem_out.at[g]).wait()

                ths = [table_v[pl.ds((g * _HG + hh) * 16, 16)]
                       for hh in range(_HG)]

                def gather_body(i, _):
                    r = i >> 4
                    cl = (i & 15) * 16
                    d = idx_v[r, pl.ds(cl, 16)]
                    k = jnp.minimum(jnp.maximum(d + 1, 0), _NVALS - 1)
                    for hh in range(_HG):
                        out_v[g * _HG + hh, r, pl.ds(cl, 16)] = (
                            ths[hh].at[k].get(mode="promise_in_bounds"))
                    return 0

                lax.fori_loop(0, _CHUNK // 16, gather_body, 0, unroll=2)

                pltpu.async_copy(
                    out_v.at[pl.ds(g * _HG, _HG)],
                    out_hbm.at[pl.ds(wid * _H + g * _HG, _HG),
                               pl.ds(c * _R, _R), :],
                    sem_out.at[g])

            nxt = jnp.minimum(c + 2, _NSTRIP - 1)
            pltpu.async_copy(
                dist_hbm.at[wid, pl.ds(nxt * _R, _R), :], idx_v, sem_in)
        return 0

    lax.fori_loop(0, _NSTRIP // 2, pair_body, 0)

    for g in range(_NG):
        pltpu.make_async_copy(
            out_v.at[pl.ds(g * _HG, _HG)],
            out_hbm.at[pl.ds(0, _HG), pl.ds(0, _R), :],
            sem_out.at[g]).wait()
    for p in range(2):
        pltpu.make_async_copy(
            dist_hbm.at[0, pl.ds(0, _R), :], idx_bufs[p], sem_ins[p]).wait()


def kernel(dist_matrix, bias_embedding):
    B, N, _ = dist_matrix.shape
    table_flat = jnp.pad(bias_embedding.T, ((0, 0), (0, 4))).reshape(_H * 16)

    mesh = plsc.VectorSubcoreMesh(core_axis_name="c", subcore_axis_name="s")
    k = functools.partial(
        pl.kernel,
        mesh=mesh,
        out_type=jax.ShapeDtypeStruct((B * _H, N, N), jnp.float32),
        scratch_types=[
            pltpu.VMEM((_H * 16,), jnp.float32),
            pltpu.VMEM((_R, _N), jnp.int32),
            pltpu.VMEM((_R, _N), jnp.int32),
            pltpu.VMEM((_H, _R, _N), jnp.float32),
            pltpu.SemaphoreType.DMA,
            pltpu.SemaphoreType.DMA,
            pltpu.SemaphoreType.DMA((_NG,)),
        ],
        compiler_params=pltpu.CompilerParams(
            needs_layout_passes=False, use_tc_tiling_on_sc=True),
    )(_sc_body)
    return k(dist_matrix, table_flat)
